# baseline (speedup 1.0000x reference)
import functools

import jax
import jax.numpy as jnp
from jax import lax
from jax.experimental import pallas as pl
from jax.experimental.pallas import tpu as pltpu
from jax.experimental.pallas import tpu_sc as plsc

_EPS = 1e-5
_NC = 2
_NS = 16


def _make_deg_kernel(E, n_pad):
    per_tile = E // (_NC * _NS)
    n_per_tile = n_pad // _NS
    full_iters = per_tile // 16
    tail = per_tile - full_iters * 16
    n_iters = full_iters + (1 if tail else 0)
    buf_len = n_iters * 16

    mesh = plsc.VectorSubcoreMesh(core_axis_name="c", subcore_axis_name="s")

    @functools.partial(
        pl.kernel,
        mesh=mesh,
        out_type=jax.ShapeDtypeStruct((_NC, n_pad), jnp.float32),
        scratch_types=[
            pltpu.VMEM((buf_len,), jnp.int32),
            pltpu.VMEM((n_pad,), jnp.float32),
            pltpu.VMEM((_NS, n_per_tile), jnp.float32),
            pltpu.VMEM_SHARED((_NS, n_pad), jnp.float32),
        ],
        compiler_params=pltpu.CompilerParams(needs_layout_passes=False),
    )
    def deg_kernel(dst_hbm, out_hbm, dstbuf, hist, colbuf, shared):
        core = lax.axis_index("c")
        sub = lax.axis_index("s")
        base = core * (_NS * per_tile) + sub * per_tile
        pltpu.sync_copy(dst_hbm.at[pl.ds(base, per_tile)],
                        dstbuf.at[pl.ds(0, per_tile)])

        zeros16 = jnp.zeros((16,), jnp.float32)

        def zbody(i, _):
            hist[pl.ds(i * 16, 16)] = zeros16
            return 0

        lax.fori_loop(0, n_pad // 16, zbody, 0, unroll=8)

        ones16 = jnp.ones((16,), jnp.float32)
        iota16 = lax.iota(jnp.int32, 16)

        def hbody(i, _):
            idx = dstbuf[pl.ds(i * 16, 16)]
            lim = jnp.where(i < full_iters, 16, tail)
            plsc.addupdate_scatter(hist, [idx], ones16, mask=iota16 < lim)
            return 0

        lax.fori_loop(0, n_iters, hbody, 0)

        pltpu.sync_copy(hist, shared.at[sub])
        plsc.subcore_barrier()
        pltpu.sync_copy(shared.at[:, pl.ds(sub * n_per_tile, n_per_tile)], colbuf)

        def sbody(j, _):
            s = colbuf[0, pl.ds(j * 16, 16)]
            for r in range(1, _NS):
                s = s + colbuf[r, pl.ds(j * 16, 16)]
            colbuf[0, pl.ds(j * 16, 16)] = s
            return 0

        lax.fori_loop(0, n_per_tile // 16, sbody, 0)
        pltpu.sync_copy(colbuf.at[0],
                        out_hbm.at[core, pl.ds(sub * n_per_tile, n_per_tile)])

    return deg_kernel


def _make_agg_kernel(N, E, n_pad, half, n_batches, batch, ring=4):
    RING = ring
    IRING = 2 * ring
    n_per_tile = n_pad // _NS
    per_tile = E // _NS
    n_groups = (n_batches - 1) // IRING
    rest = list(range(n_groups * IRING + 1, n_batches))

    mesh = plsc.VectorSubcoreMesh(core_axis_name="c", subcore_axis_name="s")

    @functools.partial(
        pl.kernel,
        mesh=mesh,
        out_type=jax.ShapeDtypeStruct((_NC, n_pad, half), jnp.float32),
        scratch_types=(
            [pltpu.VMEM((batch,), jnp.int32)] * IRING
            + [pltpu.VMEM((batch,), jnp.int32)] * IRING
            + [pltpu.VMEM((batch, half), jnp.float32)] * RING
            + [pltpu.VMEM_SHARED((n_pad, half), jnp.float32)]
            + [pltpu.SemaphoreType.DMA] * (IRING + 2 * RING)
        ),
    )
    def agg_kernel(srcp_hbm, dstp_hbm, hs_hbm, out_hbm, *sc):
        sidx = sc[0:IRING]
        didx = sc[IRING:2 * IRING]
        rows = sc[2 * IRING:2 * IRING + RING]
        acc = sc[2 * IRING + RING]
        semi = sc[2 * IRING + RING + 1:2 * IRING + RING + 1 + IRING]
        semg = sc[3 * IRING + RING + 1:3 * IRING + 2 * RING + 1]
        sems = sc[3 * IRING + 2 * RING + 1:3 * IRING + 3 * RING + 1]
        core = lax.axis_index("c")
        sub = lax.axis_index("s")
        tbase = sub * per_tile

        pltpu.sync_copy(
            hs_hbm.at[pl.ds(core * n_pad + sub * n_per_tile, n_per_tile)],
            acc.at[pl.ds(sub * n_per_tile, n_per_tile)])

        def idx_start(j, q):
            off = tbase + j * batch
            pltpu.async_copy(srcp_hbm.at[pl.ds(core * E + off, batch)],
                             sidx[q], semi[q])
            pltpu.async_copy(dstp_hbm.at[pl.ds(off, batch)], didx[q], semi[q])

        def idx_wait(q):
            pltpu.make_async_copy(dstp_hbm.at[pl.ds(0, batch)], sidx[q],
                                  semi[q]).wait()
            pltpu.make_async_copy(dstp_hbm.at[pl.ds(0, batch)], didx[q],
                                  semi[q]).wait()

        def gather_start(b, q):
            pltpu.async_copy(hs_hbm.at[sidx[q]], rows[b], semg[b])

        def gather_wait(b):
            pltpu.make_async_copy(hs_hbm.at[sidx[0]], rows[b], semg[b]).wait()

        def scatter_start(b, q):
            pltpu.async_copy(rows[b], acc.at[didx[q]], sems[b], add=True)

        def scatter_wait(b):
            pltpu.make_async_copy(rows[b], acc.at[didx[0]], sems[b]).wait()

        plsc.subcore_barrier()

        for k in range(IRING - 1):
            idx_start(k, k)
        for k in range(RING - 1):
            idx_wait(k)
            gather_start(k, k)
        gather_wait(0)
        scatter_start(0, 0)
        idx_wait(RING - 1)
        gather_start(RING - 1, RING - 1)
        idx_start(IRING - 1, IRING - 1)

        def step(j, b, q):
            bp = (b + RING - 1) % RING
            gather_wait(b)
            scatter_start(b, q)

            @pl.when(j + RING - 1 < n_batches)
            def _():
                scatter_wait(bp)
                idx_wait((q + RING - 1) % IRING)
                gather_start(bp, (q + RING - 1) % IRING)

            @pl.when(j + IRING - 1 < n_batches)
            def _():
                idx_start(j + IRING - 1, (q + IRING - 1) % IRING)

        def group_body(g, _):
            for off in range(IRING):
                j = g * IRING + 1 + off
                step(j, (1 + off) % RING, (1 + off) % IRING)
            return 0

        lax.fori_loop(0, n_groups, group_body, 0)

        for j in rest:
            step(j, j % RING, j % IRING)

        for j in range(n_batches - RING, n_batches):
            scatter_wait(j % RING)

        plsc.subcore_barrier()
        pltpu.sync_copy(acc.at[pl.ds(sub * n_per_tile, n_per_tile)],
                        out_hbm.at[core, pl.ds(sub * n_per_tile, n_per_tile)])

    return agg_kernel


def _make_bn_kernel(N, C, n_pad):
    half = C // 2

    def body(x_ref, w_ref, b_ref, g_ref, be_ref, degc_ref, out_ref):
        X = x_ref[...]
        W = w_ref[...]
        H = lax.dot_general(X, W, (((1,), (1,)), ((), ())),
                            preferred_element_type=jnp.float32)
        H = H + b_ref[...]
        mean = jnp.mean(H, axis=0, keepdims=True)
        var = jnp.mean(H * H, axis=0, keepdims=True) - mean * mean
        Hn = g_ref[...] * (H - mean) * lax.rsqrt(var + _EPS) + be_ref[...]
        deg = degc_ref[pl.ds(0, N), 0:1] + degc_ref[pl.ds(0, N), 1:2] + 1.0
        dinv = lax.rsqrt(deg)
        Hs = Hn * dinv
        out_ref[pl.ds(0, N), :] = Hs[:, :half]
        out_ref[pl.ds(n_pad, N), :] = Hs[:, half:]

    return pl.pallas_call(
        body,
        out_shape=jax.ShapeDtypeStruct((2 * n_pad, half), jnp.float32),
    )


def _make_out_kernel(N, C, n_pad):
    half = C // 2

    def body(acc_ref, degc_ref, out_ref):
        deg = degc_ref[pl.ds(0, N), 0:1] + degc_ref[pl.ds(0, N), 1:2] + 1.0
        dinv = lax.rsqrt(deg)
        a0 = acc_ref[0, pl.ds(0, N), :]
        a1 = acc_ref[1, pl.ds(0, N), :]
        out_ref[:, 0:half] = jnp.maximum(a0 * dinv, 0.0)
        out_ref[:, half:C] = jnp.maximum(a1 * dinv, 0.0)

    return pl.pallas_call(
        body,
        out_shape=jax.ShapeDtypeStruct((N, C), jnp.float32),
    )


def kernel(X, edge_index, W, b, gamma, beta):
    N, C = X.shape
    E = edge_index.shape[1]
    half = C // 2
    n_per_tile = -(-N // (_NS * 16)) * 16
    n_pad = n_per_tile * _NS

    src = edge_index[0]
    dst = edge_index[1]

    degp = _make_deg_kernel(E, n_pad)(dst)
    degc = jnp.transpose(degp)
    b2 = jnp.reshape(b, (1, C))
    g2 = jnp.reshape(gamma, (1, C))
    be2 = jnp.reshape(beta, (1, C))
    hs = _make_bn_kernel(N, C, n_pad)(X, W, b2, g2, be2, degc)

    per_tile = E // _NS
    batch = 40
    n_batches = per_tile // batch
    srcp = jnp.concatenate([src, src + n_pad])

    acc = _make_agg_kernel(N, E, n_pad, half, n_batches, batch, ring=6)(
        srcp, dst, hs)
    out = _make_out_kernel(N, C, n_pad)(acc, degc)
    return out


# speedup vs baseline: 1.0018x; 1.0018x over previous
import functools

import jax
import jax.numpy as jnp
from jax import lax
from jax.experimental import pallas as pl
from jax.experimental.pallas import tpu as pltpu
from jax.experimental.pallas import tpu_sc as plsc

_EPS = 1e-5
_NC = 2
_NS = 16


def _make_deg_kernel(E, n_pad):
    per_tile = E // (_NC * _NS)
    n_per_tile = n_pad // _NS
    full_iters = per_tile // 16
    tail = per_tile - full_iters * 16
    n_iters = full_iters + (1 if tail else 0)
    buf_len = n_iters * 16

    mesh = plsc.VectorSubcoreMesh(core_axis_name="c", subcore_axis_name="s")

    @functools.partial(
        pl.kernel,
        mesh=mesh,
        out_type=jax.ShapeDtypeStruct((_NC, n_pad), jnp.float32),
        scratch_types=[
            pltpu.VMEM((buf_len,), jnp.int32),
            pltpu.VMEM((n_pad,), jnp.float32),
            pltpu.VMEM((_NS, n_per_tile), jnp.float32),
            pltpu.VMEM_SHARED((_NS, n_pad), jnp.float32),
        ],
        compiler_params=pltpu.CompilerParams(needs_layout_passes=False),
    )
    def deg_kernel(dst_hbm, out_hbm, dstbuf, hist, colbuf, shared):
        core = lax.axis_index("c")
        sub = lax.axis_index("s")
        base = core * (_NS * per_tile) + sub * per_tile
        pltpu.sync_copy(dst_hbm.at[pl.ds(base, per_tile)],
                        dstbuf.at[pl.ds(0, per_tile)])

        zeros16 = jnp.zeros((16,), jnp.float32)

        def zbody(i, _):
            hist[pl.ds(i * 16, 16)] = zeros16
            return 0

        lax.fori_loop(0, n_pad // 16, zbody, 0, unroll=8)

        ones16 = jnp.ones((16,), jnp.float32)
        iota16 = lax.iota(jnp.int32, 16)

        def hbody(i, _):
            idx = dstbuf[pl.ds(i * 16, 16)]
            lim = jnp.where(i < full_iters, 16, tail)
            plsc.addupdate_scatter(hist, [idx], ones16, mask=iota16 < lim)
            return 0

        lax.fori_loop(0, n_iters, hbody, 0)

        pltpu.sync_copy(hist, shared.at[sub])
        plsc.subcore_barrier()
        pltpu.sync_copy(shared.at[:, pl.ds(sub * n_per_tile, n_per_tile)], colbuf)

        def sbody(j, _):
            s = colbuf[0, pl.ds(j * 16, 16)]
            for r in range(1, _NS):
                s = s + colbuf[r, pl.ds(j * 16, 16)]
            colbuf[0, pl.ds(j * 16, 16)] = s
            return 0

        lax.fori_loop(0, n_per_tile // 16, sbody, 0)
        pltpu.sync_copy(colbuf.at[0],
                        out_hbm.at[core, pl.ds(sub * n_per_tile, n_per_tile)])

    return deg_kernel


def _make_agg_kernel(N, E, n_pad, half, n_batches, batch, ring=4):
    RING = ring
    IRING = 2 * ring
    n_per_tile = n_pad // _NS
    per_tile = E // _NS
    n_groups = (n_batches - 1) // IRING
    rest = list(range(n_groups * IRING + 1, n_batches))

    mesh = plsc.VectorSubcoreMesh(core_axis_name="c", subcore_axis_name="s")

    @functools.partial(
        pl.kernel,
        mesh=mesh,
        out_type=jax.ShapeDtypeStruct((_NC, n_pad, half), jnp.float32),
        scratch_types=(
            [pltpu.VMEM((batch,), jnp.int32)] * IRING
            + [pltpu.VMEM((batch,), jnp.int32)] * IRING
            + [pltpu.VMEM((batch, half), jnp.float32)] * RING
            + [pltpu.VMEM_SHARED((n_pad, half), jnp.float32)]
            + [pltpu.SemaphoreType.DMA] * (IRING + 2 * RING)
        ),
    )
    def agg_kernel(srcp_hbm, dstp_hbm, hs_hbm, out_hbm, *sc):
        sidx = sc[0:IRING]
        didx = sc[IRING:2 * IRING]
        rows = sc[2 * IRING:2 * IRING + RING]
        acc = sc[2 * IRING + RING]
        semi = sc[2 * IRING + RING + 1:2 * IRING + RING + 1 + IRING]
        semg = sc[3 * IRING + RING + 1:3 * IRING + 2 * RING + 1]
        sems = sc[3 * IRING + 2 * RING + 1:3 * IRING + 3 * RING + 1]
        core = lax.axis_index("c")
        sub = lax.axis_index("s")
        tbase = sub * per_tile

        pltpu.sync_copy(
            hs_hbm.at[pl.ds(core * n_pad + sub * n_per_tile, n_per_tile)],
            acc.at[pl.ds(sub * n_per_tile, n_per_tile)])

        def idx_start(j, q):
            off = tbase + j * batch
            pltpu.async_copy(srcp_hbm.at[pl.ds(core * E + off, batch)],
                             sidx[q], semi[q])
            pltpu.async_copy(dstp_hbm.at[pl.ds(off, batch)], didx[q], semi[q])

        def idx_wait(q):
            pltpu.make_async_copy(dstp_hbm.at[pl.ds(0, batch)], sidx[q],
                                  semi[q]).wait()
            pltpu.make_async_copy(dstp_hbm.at[pl.ds(0, batch)], didx[q],
                                  semi[q]).wait()

        def gather_start(b, q):
            pltpu.async_copy(hs_hbm.at[sidx[q]], rows[b], semg[b])

        def gather_wait(b):
            pltpu.make_async_copy(hs_hbm.at[sidx[0]], rows[b], semg[b]).wait()

        def scatter_start(b, q):
            pltpu.async_copy(rows[b], acc.at[didx[q]], sems[b], add=True)

        def scatter_wait(b):
            pltpu.make_async_copy(rows[b], acc.at[didx[0]], sems[b]).wait()

        plsc.subcore_barrier()

        for k in range(IRING - 1):
            idx_start(k, k)
        for k in range(RING - 1):
            idx_wait(k)
            gather_start(k, k)
        gather_wait(0)
        scatter_start(0, 0)
        idx_wait(RING - 1)
        gather_start(RING - 1, RING - 1)
        idx_start(IRING - 1, IRING - 1)

        def step(j, b, q):
            bp = (b + RING - 1) % RING
            gather_wait(b)
            scatter_start(b, q)

            @pl.when(j + RING - 1 < n_batches)
            def _():
                scatter_wait(bp)
                idx_wait((q + RING - 1) % IRING)
                gather_start(bp, (q + RING - 1) % IRING)

            @pl.when(j + IRING - 1 < n_batches)
            def _():
                idx_start(j + IRING - 1, (q + IRING - 1) % IRING)

        def group_body(g, _):
            for off in range(IRING):
                j = g * IRING + 1 + off
                step(j, (1 + off) % RING, (1 + off) % IRING)
            return 0

        lax.fori_loop(0, n_groups, group_body, 0)

        for j in rest:
            step(j, j % RING, j % IRING)

        for j in range(n_batches - RING, n_batches):
            scatter_wait(j % RING)

        plsc.subcore_barrier()
        pltpu.sync_copy(acc.at[pl.ds(sub * n_per_tile, n_per_tile)],
                        out_hbm.at[core, pl.ds(sub * n_per_tile, n_per_tile)])

    return agg_kernel


def _make_bn_kernel(N, C, n_pad):
    half = C // 2

    def body(x_ref, w_ref, b_ref, g_ref, be_ref, degc_ref, out_ref):
        X = x_ref[...]
        W = w_ref[...]
        H = lax.dot_general(X, W, (((1,), (1,)), ((), ())),
                            preferred_element_type=jnp.float32)
        H = H + b_ref[...]
        mean = jnp.mean(H, axis=0, keepdims=True)
        var = jnp.mean(H * H, axis=0, keepdims=True) - mean * mean
        Hn = g_ref[...] * (H - mean) * lax.rsqrt(var + _EPS) + be_ref[...]
        deg = degc_ref[pl.ds(0, N), 0:1] + degc_ref[pl.ds(0, N), 1:2] + 1.0
        dinv = lax.rsqrt(deg)
        Hs = Hn * dinv
        out_ref[pl.ds(0, N), :] = Hs[:, :half]
        out_ref[pl.ds(n_pad, N), :] = Hs[:, half:]

    return pl.pallas_call(
        body,
        out_shape=jax.ShapeDtypeStruct((2 * n_pad, half), jnp.float32),
    )


def _make_out_kernel(N, C, n_pad):
    half = C // 2

    blk = 2000

    def body(acc_ref, degc_ref, out_ref):
        deg = degc_ref[:, 0:1] + degc_ref[:, 1:2] + 1.0
        dinv = lax.rsqrt(deg)
        out_ref[:, 0:half] = jnp.maximum(acc_ref[0] * dinv, 0.0)
        out_ref[:, half:C] = jnp.maximum(acc_ref[1] * dinv, 0.0)

    return pl.pallas_call(
        body,
        grid=(N // blk,),
        in_specs=[
            pl.BlockSpec((_NC, blk, half), lambda i: (0, i, 0)),
            pl.BlockSpec((blk, 2), lambda i: (i, 0)),
        ],
        out_specs=pl.BlockSpec((blk, C), lambda i: (i, 0)),
        out_shape=jax.ShapeDtypeStruct((N, C), jnp.float32),
    )


def kernel(X, edge_index, W, b, gamma, beta):
    N, C = X.shape
    E = edge_index.shape[1]
    half = C // 2
    n_per_tile = -(-N // (_NS * 16)) * 16
    n_pad = n_per_tile * _NS

    src = edge_index[0]
    dst = edge_index[1]

    degp = _make_deg_kernel(E, n_pad)(dst)
    degc = jnp.transpose(degp)
    b2 = jnp.reshape(b, (1, C))
    g2 = jnp.reshape(gamma, (1, C))
    be2 = jnp.reshape(beta, (1, C))
    hs = _make_bn_kernel(N, C, n_pad)(X, W, b2, g2, be2, degc)

    per_tile = E // _NS
    batch = 40
    n_batches = per_tile // batch
    srcp = jnp.concatenate([src, src + n_pad])

    acc = _make_agg_kernel(N, E, n_pad, half, n_batches, batch, ring=6)(
        srcp, dst, hs)
    out = _make_out_kernel(N, C, n_pad)(acc, degc)
    return out
